# baseline (speedup 1.0000x reference)
import functools

import jax
import jax.numpy as jnp
import numpy as np
from jax import lax
from jax.experimental import pallas as pl
from jax.experimental.pallas import tpu as pltpu
from jax.experimental.pallas import tpu_sc as plsc

N = 10000
E = 320000
ND = 128
RAD = 16
ANG = 4
SW = 148
SWP = 160
UW = 276
UWP = 288
D1, D2, H = 90, 32, 64

NC, NS = 2, 16
NW = NC * NS
EPW = E // NW
CHM = 80
NCHM = EPW // CHM
CHU = 40
NCHU = EPW // CHU
NCHU2 = NCHU // 2
APAD = NW * 320
ZROWS = APAD // NS
RPT = N // NS


def _sig(x):
    return 1.0 / (1.0 + jnp.exp(-x))


_PERM = np.zeros(UWP, np.int32)
for _m in range(UWP // 32):
    for _k in range(16):
        _PERM[32 * _m + 2 * _k] = 32 * _m + _k
        _PERM[32 * _m + 2 * _k + 1] = 32 * _m + 16 + _k


def _pack_bf16(x):
    return jax.lax.bitcast_convert_type(
        x.reshape(x.shape[:-1] + (x.shape[-1] // 2, 2)), jnp.int32)


def _tables_body(nf_ref, w1g_ref, wes_ref, wed_ref, g_ref, p_ref, q_ref):
    nf = nf_ref[...]
    g_ref[...] = jnp.dot(nf, w1g_ref[...], preferred_element_type=jnp.float32)
    p_ref[...] = jnp.dot(nf, wes_ref[...],
                         preferred_element_type=jnp.float32).astype(jnp.bfloat16)
    q_ref[...] = jnp.dot(nf, wed_ref[...],
                         preferred_element_type=jnp.float32).astype(jnp.bfloat16)


def _tables_call(nf, w1g, wes, wed):
    return pl.pallas_call(
        _tables_body,
        out_shape=(
            jax.ShapeDtypeStruct((N, SWP), jnp.float32),
            jax.ShapeDtypeStruct((N, UWP), jnp.bfloat16),
            jax.ShapeDtypeStruct((N, UWP), jnp.bfloat16),
        ),
    )(nf, w1g, wes, wed)


BE = 4000

def _sr_body(er_ref, ea_ref, w1r_ref, w1a_ref, b1_ref, wer_ref, wea_ref,
             be_ref, s_ref, r_ref):
    er = er_ref[...]
    ea = ea_ref[...]
    s_ref[...] = (jnp.dot(er, w1r_ref[...], preferred_element_type=jnp.float32)
                  + jnp.dot(ea, w1a_ref[...], preferred_element_type=jnp.float32)
                  + b1_ref[...])
    r_ref[...] = (jnp.dot(er, wer_ref[...], preferred_element_type=jnp.float32)
                  + jnp.dot(ea, wea_ref[...], preferred_element_type=jnp.float32)
                  + be_ref[...]).astype(jnp.bfloat16)


def _sr_call(er, ea, w1r, w1a, b1p, wer, wea, bep):
    nblk = E // BE
    return pl.pallas_call(
        _sr_body,
        grid=(nblk,),
        in_specs=[
            pl.BlockSpec((BE, RAD), lambda i: (i, 0)),
            pl.BlockSpec((BE, ANG), lambda i: (i, 0)),
            pl.BlockSpec((RAD, SWP), lambda i: (0, 0)),
            pl.BlockSpec((ANG, SWP), lambda i: (0, 0)),
            pl.BlockSpec((1, SWP), lambda i: (0, 0)),
            pl.BlockSpec((RAD, UWP), lambda i: (0, 0)),
            pl.BlockSpec((ANG, UWP), lambda i: (0, 0)),
            pl.BlockSpec((1, UWP), lambda i: (0, 0)),
        ],
        out_specs=(
            pl.BlockSpec((BE, SWP), lambda i: (i, 0)),
            pl.BlockSpec((BE, UWP), lambda i: (i, 0)),
        ),
        out_shape=(
            jax.ShapeDtypeStruct((E, SWP), jnp.float32),
            jax.ShapeDtypeStruct((E, UWP), jnp.bfloat16),
        ),
    )(er, ea, w1r, w1a, b1p, wer, wea, bep)


def _node_body(msg_ref, nf_ref, w2a_ref, b2a_ref, w2b_ref, b2b_ref,
               w2c_ref, b2c_ref, w2d_ref, b2d_ref, ws_ref, bs_ref,
               wn_ref, bn_ref, out_ref):
    m = msg_ref[0][:N, :SW] + msg_ref[1][:N, :SW]
    h = jnp.maximum(jnp.dot(m, w2a_ref[...], preferred_element_type=jnp.float32)
                    + b2a_ref[...], 0.0)
    h = jnp.maximum(jnp.dot(h, w2b_ref[...], preferred_element_type=jnp.float32)
                    + b2b_ref[...], 0.0)
    h = jnp.maximum(jnp.dot(h, w2c_ref[...], preferred_element_type=jnp.float32)
                    + b2c_ref[...], 0.0)
    h = jnp.dot(h, w2d_ref[...], preferred_element_type=jnp.float32) + b2d_ref[...]
    nf = nf_ref[...]
    z = (jnp.dot(nf, ws_ref[...], preferred_element_type=jnp.float32) + bs_ref[...]
         + jnp.dot(h, wn_ref[...], preferred_element_type=jnp.float32) + bn_ref[...])
    out_ref[...] = _sig(z) + nf


def _node_call(msg, nf, w2a, b2a, w2b, b2b, w2c, b2c, w2d, b2d, ws, bs, wn, bn):
    return pl.pallas_call(
        _node_body,
        out_shape=jax.ShapeDtypeStruct((N, ND), jnp.float32),
    )(msg, nf, w2a, b2a, w2b, b2b, w2c, b2c, w2d, b2d, ws, bs, wn, bn)


_mesh = plsc.VectorSubcoreMesh(core_axis_name="c", subcore_axis_name="s")


@functools.partial(
    pl.kernel,
    out_type=jax.ShapeDtypeStruct((NC, APAD, SWP), jnp.float32),
    mesh=_mesh,
    scratch_types=[
        pltpu.VMEM((CHM,), jnp.int32),
        pltpu.VMEM((CHM,), jnp.int32),
        pltpu.VMEM((CHM,), jnp.int32),
        pltpu.VMEM((CHM, SWP), jnp.float32),
        pltpu.VMEM((CHM, SWP), jnp.float32),
        pltpu.VMEM_SHARED((APAD, SWP), jnp.float32),
        pltpu.SemaphoreType.DMA,
    ],
    compiler_params=pltpu.CompilerParams(use_tc_tiling_on_sc=False,
                                         needs_layout_passes=False),
)
def _sc_messages(src_hbm, dst_hbm, g_hbm, s_hbm, msg_hbm,
                 src_idx, dst_idx, src2_idx,
                 g_v, s_v, acc, sem_a):
    c = lax.axis_index("c")
    s = lax.axis_index("s")
    wid = c * NS + s
    base = wid * EPW

    @pl.loop(0, CHM)
    def _zrow(r):
        for j in range(SWP // 16):
            s_v[r, pl.ds(j * 16, 16)] = jnp.zeros((16,), jnp.float32)

    for z in range(ZROWS // CHM):
        zoff = pl.multiple_of(s * ZROWS + z * CHM, 8)
        pltpu.sync_copy(s_v, acc.at[pl.ds(zoff, CHM)])

    plsc.subcore_barrier()

    @pl.loop(0, NCHM)
    def _chunk(k):
        off = pl.multiple_of(base + k * CHM, 8)
        pltpu.sync_copy(src_hbm.at[pl.ds(off, CHM)], src_idx)
        pltpu.sync_copy(dst_hbm.at[pl.ds(off, CHM)], dst_idx)
        pltpu.async_copy(src_hbm.at[src_idx], src2_idx, sem_a).wait()
        cp_g = pltpu.async_copy(g_hbm.at[src2_idx], g_v, sem_a)
        cp_s = pltpu.async_copy(s_hbm.at[pl.ds(off, CHM)], s_v, sem_a)
        cp_g.wait()
        cp_s.wait()

        @plsc.parallel_loop(0, CHM, unroll=2)
        def _sub(r):
            for j in range(SWP // 16):
                x = g_v[r, pl.ds(j * 16, 16)] + s_v[r, pl.ds(j * 16, 16)]
                s_v[r, pl.ds(j * 16, 16)] = _sig(x)

        pltpu.sync_copy(s_v, acc.at[dst_idx], add=True)

    plsc.subcore_barrier()
    moff = pl.multiple_of(s * ZROWS, 8)
    pltpu.sync_copy(acc.at[pl.ds(moff, ZROWS)],
                    msg_hbm.at[c, pl.ds(moff, ZROWS)])


UWH = UWP // 2


@functools.partial(
    pl.kernel,
    out_type=jax.ShapeDtypeStruct((E, UW), jnp.float32),
    mesh=_mesh,
    scratch_types=[
        pltpu.VMEM((2, 2, CHU), jnp.int32),
        pltpu.VMEM((2, CHU, UWH), jnp.int32),
        pltpu.VMEM((2, CHU, UWH), jnp.int32),
        pltpu.VMEM((2, CHU, UWH), jnp.int32),
        pltpu.VMEM((2, CHU, UW), jnp.float32),
        pltpu.SemaphoreType.DMA,
        pltpu.SemaphoreType.DMA,
        pltpu.SemaphoreType.DMA,
        pltpu.SemaphoreType.DMA,
        pltpu.SemaphoreType.DMA,
    ],
    compiler_params=pltpu.CompilerParams(use_tc_tiling_on_sc=False,
                                         needs_layout_passes=False),
)
def _sc_edge_update(ei_hbm, p_hbm, q_hbm, r_hbm, out_hbm,
                    ei_v, p_v, q_v, r_v, o_v,
                    sem_i, sem_g0, sem_g1, sem_o0, sem_o1):
    c = lax.axis_index("c")
    s = lax.axis_index("s")
    base = (c * NS + s) * EPW

    def coff(j):
        jm = jnp.minimum(j, NCHU - 1)
        return pl.multiple_of(base + jm * CHU, 8)

    def issue_idx(j, b):
        pltpu.async_copy(ei_hbm.at[:, pl.ds(coff(j), CHU)], ei_v.at[b], sem_i)

    def wait_idx(b):
        pltpu.make_async_copy(ei_hbm.at[:, pl.ds(coff(0), CHU)],
                              ei_v.at[b], sem_i).wait()

    def issue_gathers(j, b, sem):
        pltpu.async_copy(p_hbm.at[ei_v.at[b, 0]], p_v.at[b], sem)
        pltpu.async_copy(q_hbm.at[ei_v.at[b, 1]], q_v.at[b], sem)
        pltpu.async_copy(r_hbm.at[pl.ds(coff(j), CHU)], r_v.at[b], sem)

    def wait_gathers(b, sem):
        for dst in (p_v, q_v, r_v):
            pltpu.make_async_copy(r_hbm.at[pl.ds(coff(0), CHU)],
                                  dst.at[b], sem).wait()

    def compute(b):
        lane = lax.iota(jnp.int32, 16)
        tail_mask = lane < 4
        himask = jnp.full((16,), jnp.int32(-65536))

        def halves(v_ref, r, j):
            x = v_ref[b, r, pl.ds(j * 16, 16)]
            lo = plsc.bitcast(lax.shift_left(x, 16), jnp.float32)
            hi = plsc.bitcast(lax.bitwise_and(x, himask), jnp.float32)
            return lo, hi

        @plsc.parallel_loop(0, CHU, unroll=2)
        def _upd(r):
            for j in range(UWP // 32):
                plo, phi = halves(p_v, r, j)
                qlo, qhi = halves(q_v, r, j)
                rlo, rhi = halves(r_v, r, j)
                ylo = _sig(plo + qlo + rlo)
                yhi = _sig(phi + qhi + rhi)
                o_v[b, r, pl.ds(j * 32, 16)] = ylo
                if j < UWP // 32 - 1:
                    o_v[b, r, pl.ds(j * 32 + 16, 16)] = yhi
                else:
                    plsc.store_scatter(
                        o_v.at[b], [lane * 0 + r, j * 32 + 16 + lane],
                        yhi, mask=tail_mask)

    def issue_out(j, b, sem):
        pltpu.async_copy(o_v.at[b], out_hbm.at[pl.ds(coff(j), CHU)], sem)

    def wait_out(b, sem):
        pltpu.make_async_copy(o_v.at[b],
                              out_hbm.at[pl.ds(coff(0), CHU)], sem).wait()

    issue_idx(0, 0)
    wait_idx(0)
    issue_gathers(0, 0, sem_g0)
    issue_idx(1, 1)

    @pl.loop(0, NCHU2)
    def _pair(kk):
        j0 = kk * 2
        wait_idx(1)
        wait_gathers(0, sem_g0)
        issue_gathers(j0 + 1, 1, sem_g1)
        issue_idx(j0 + 2, 0)

        @pl.when(kk > 0)
        def _wo0():
            wait_out(0, sem_o0)

        compute(0)
        issue_out(j0, 0, sem_o0)

        wait_idx(0)
        wait_gathers(1, sem_g1)
        issue_gathers(j0 + 2, 0, sem_g0)
        issue_idx(j0 + 3, 1)

        @pl.when(kk > 0)
        def _wo1():
            wait_out(1, sem_o1)

        compute(1)
        issue_out(j0 + 1, 1, sem_o1)

    wait_idx(1)
    wait_gathers(0, sem_g0)
    wait_out(0, sem_o0)
    wait_out(1, sem_o1)


def kernel(node_features, edge_radial, edge_angular, edge_index,
           W1, b1, W2a, b2a, W2b, b2b, W2c, b2c, W2d, b2d,
           Ws, bs, Wn, bn, We, be):
    src = edge_index[0].astype(jnp.int32)
    dst = edge_index[1].astype(jnp.int32)

    zpad_s = jnp.zeros((SWP - SW,), jnp.float32)
    w1g = jnp.pad(W1[:ND], ((0, 0), (0, SWP - SW)))
    w1r = jnp.pad(W1[ND:ND + RAD], ((0, 0), (0, SWP - SW)))
    w1a = jnp.pad(W1[ND + RAD:], ((0, 0), (0, SWP - SW)))
    b1p = jnp.concatenate([b1, zpad_s])[None, :]
    wes = jnp.pad(We[:ND], ((0, 0), (0, UWP - UW)))[:, _PERM]
    wed = jnp.pad(We[ND:2 * ND], ((0, 0), (0, UWP - UW)))[:, _PERM]
    wer = jnp.pad(We[2 * ND:2 * ND + RAD], ((0, 0), (0, UWP - UW)))[:, _PERM]
    wea = jnp.pad(We[2 * ND + RAD:], ((0, 0), (0, UWP - UW)))[:, _PERM]
    bep = jnp.pad(be, (0, UWP - UW))[_PERM][None, :]

    g, p, q = _tables_call(node_features, w1g, wes, wed)
    s_t, r_t = _sr_call(edge_radial, edge_angular, w1r, w1a, b1p, wer, wea, bep)

    ei = edge_index.astype(jnp.int32)
    msg = _sc_messages(src, dst, g, s_t)
    edge_updated = _sc_edge_update(ei, _pack_bf16(p), _pack_bf16(q),
                                   _pack_bf16(r_t))

    node_updated = _node_call(
        msg, node_features, W2a, b2a[None, :], W2b, b2b[None, :],
        W2c, b2c[None, :], W2d, b2d[None, :], Ws, bs[None, :], Wn, bn[None, :])
    return (node_updated, edge_updated)


# speedup vs baseline: 1.5412x; 1.5412x over previous
import functools

import jax
import jax.numpy as jnp
import numpy as np
from jax import lax
from jax.experimental import pallas as pl
from jax.experimental.pallas import tpu as pltpu
from jax.experimental.pallas import tpu_sc as plsc

N = 10000
E = 320000
ND = 128
RAD = 16
ANG = 4
SW = 148
SWP = 160
UW = 276
UWP = 288
D1, D2, H = 90, 32, 64

NC, NS = 2, 16
NW = NC * NS
EPW = E // NW
CHM = 80
NCHM = EPW // CHM
CHU = 40
NCHU = EPW // CHU
NCHU2 = NCHU // 2
APAD = NW * 320
ZROWS = APAD // NS
RPT = N // NS


def _sig(x):
    return 1.0 / (1.0 + jnp.exp(-x))


_PERM = np.zeros(UWP, np.int32)
for _m in range(UWP // 32):
    for _k in range(16):
        _PERM[32 * _m + 2 * _k] = 32 * _m + _k
        _PERM[32 * _m + 2 * _k + 1] = 32 * _m + 16 + _k


def _tables_body(nf_ref, w1g_ref, wes_ref, wed_ref, g_ref, p_ref, q_ref):
    nf = nf_ref[...]
    g_ref[...] = jnp.dot(nf, w1g_ref[...], preferred_element_type=jnp.float32)
    p_ref[...] = jnp.dot(nf, wes_ref[...],
                         preferred_element_type=jnp.float32).astype(jnp.bfloat16)
    q_ref[...] = jnp.dot(nf, wed_ref[...],
                         preferred_element_type=jnp.float32).astype(jnp.bfloat16)


def _tables_call(nf, w1g, wes, wed):
    return pl.pallas_call(
        _tables_body,
        out_shape=(
            jax.ShapeDtypeStruct((N, SWP), jnp.float32),
            jax.ShapeDtypeStruct((N, UWP), jnp.bfloat16),
            jax.ShapeDtypeStruct((N, UWP), jnp.bfloat16),
        ),
    )(nf, w1g, wes, wed)


BE = 4000

def _sr_body(er_ref, ea_ref, w1r_ref, w1a_ref, b1_ref, wer_ref, wea_ref,
             be_ref, s_ref, r_ref):
    er = er_ref[...]
    ea = ea_ref[...]
    s_ref[...] = (jnp.dot(er, w1r_ref[...], preferred_element_type=jnp.float32)
                  + jnp.dot(ea, w1a_ref[...], preferred_element_type=jnp.float32)
                  + b1_ref[...])
    r_ref[...] = (jnp.dot(er, wer_ref[...], preferred_element_type=jnp.float32)
                  + jnp.dot(ea, wea_ref[...], preferred_element_type=jnp.float32)
                  + be_ref[...]).astype(jnp.bfloat16)


def _sr_call(er, ea, w1r, w1a, b1p, wer, wea, bep):
    nblk = E // BE
    return pl.pallas_call(
        _sr_body,
        grid=(nblk,),
        in_specs=[
            pl.BlockSpec((BE, RAD), lambda i: (i, 0)),
            pl.BlockSpec((BE, ANG), lambda i: (i, 0)),
            pl.BlockSpec((RAD, SWP), lambda i: (0, 0)),
            pl.BlockSpec((ANG, SWP), lambda i: (0, 0)),
            pl.BlockSpec((1, SWP), lambda i: (0, 0)),
            pl.BlockSpec((RAD, UWP), lambda i: (0, 0)),
            pl.BlockSpec((ANG, UWP), lambda i: (0, 0)),
            pl.BlockSpec((1, UWP), lambda i: (0, 0)),
        ],
        out_specs=(
            pl.BlockSpec((BE, SWP), lambda i: (i, 0)),
            pl.BlockSpec((BE, UWP), lambda i: (i, 0)),
        ),
        out_shape=(
            jax.ShapeDtypeStruct((E, SWP), jnp.float32),
            jax.ShapeDtypeStruct((E, UWP), jnp.bfloat16),
        ),
    )(er, ea, w1r, w1a, b1p, wer, wea, bep)


def _node_body(msg_ref, nf_ref, w2a_ref, b2a_ref, w2b_ref, b2b_ref,
               w2c_ref, b2c_ref, w2d_ref, b2d_ref, ws_ref, bs_ref,
               wn_ref, bn_ref, out_ref):
    m = msg_ref[0][:N, :SW] + msg_ref[1][:N, :SW]
    h = jnp.maximum(jnp.dot(m, w2a_ref[...], preferred_element_type=jnp.float32)
                    + b2a_ref[...], 0.0)
    h = jnp.maximum(jnp.dot(h, w2b_ref[...], preferred_element_type=jnp.float32)
                    + b2b_ref[...], 0.0)
    h = jnp.maximum(jnp.dot(h, w2c_ref[...], preferred_element_type=jnp.float32)
                    + b2c_ref[...], 0.0)
    h = jnp.dot(h, w2d_ref[...], preferred_element_type=jnp.float32) + b2d_ref[...]
    nf = nf_ref[...]
    z = (jnp.dot(nf, ws_ref[...], preferred_element_type=jnp.float32) + bs_ref[...]
         + jnp.dot(h, wn_ref[...], preferred_element_type=jnp.float32) + bn_ref[...])
    out_ref[...] = _sig(z) + nf


def _node_call(msg, nf, w2a, b2a, w2b, b2b, w2c, b2c, w2d, b2d, ws, bs, wn, bn):
    return pl.pallas_call(
        _node_body,
        out_shape=jax.ShapeDtypeStruct((N, ND), jnp.float32),
    )(msg, nf, w2a, b2a, w2b, b2b, w2c, b2c, w2d, b2d, ws, bs, wn, bn)


_mesh = plsc.VectorSubcoreMesh(core_axis_name="c", subcore_axis_name="s")


@functools.partial(
    pl.kernel,
    out_type=jax.ShapeDtypeStruct((NC, APAD, SWP), jnp.float32),
    mesh=_mesh,
    scratch_types=[
        pltpu.VMEM((CHM,), jnp.int32),
        pltpu.VMEM((CHM,), jnp.int32),
        pltpu.VMEM((CHM,), jnp.int32),
        pltpu.VMEM((CHM, SWP), jnp.float32),
        pltpu.VMEM((CHM, SWP), jnp.float32),
        pltpu.VMEM_SHARED((APAD, SWP), jnp.float32),
        pltpu.SemaphoreType.DMA,
    ],
    compiler_params=pltpu.CompilerParams(use_tc_tiling_on_sc=False,
                                         needs_layout_passes=False),
)
def _sc_messages(src_hbm, dst_hbm, g_hbm, s_hbm, msg_hbm,
                 src_idx, dst_idx, src2_idx,
                 g_v, s_v, acc, sem_a):
    c = lax.axis_index("c")
    s = lax.axis_index("s")
    wid = c * NS + s
    base = wid * EPW

    @pl.loop(0, CHM)
    def _zrow(r):
        for j in range(SWP // 16):
            s_v[r, pl.ds(j * 16, 16)] = jnp.zeros((16,), jnp.float32)

    for z in range(ZROWS // CHM):
        zoff = pl.multiple_of(s * ZROWS + z * CHM, 8)
        pltpu.sync_copy(s_v, acc.at[pl.ds(zoff, CHM)])

    plsc.subcore_barrier()

    @pl.loop(0, NCHM)
    def _chunk(k):
        off = pl.multiple_of(base + k * CHM, 8)
        pltpu.sync_copy(src_hbm.at[pl.ds(off, CHM)], src_idx)
        pltpu.sync_copy(dst_hbm.at[pl.ds(off, CHM)], dst_idx)
        pltpu.async_copy(src_hbm.at[src_idx], src2_idx, sem_a).wait()
        cp_g = pltpu.async_copy(g_hbm.at[src2_idx], g_v, sem_a)
        cp_s = pltpu.async_copy(s_hbm.at[pl.ds(off, CHM)], s_v, sem_a)
        cp_g.wait()
        cp_s.wait()

        @plsc.parallel_loop(0, CHM, unroll=2)
        def _sub(r):
            for j in range(SWP // 16):
                x = g_v[r, pl.ds(j * 16, 16)] + s_v[r, pl.ds(j * 16, 16)]
                s_v[r, pl.ds(j * 16, 16)] = _sig(x)

        pltpu.sync_copy(s_v, acc.at[dst_idx], add=True)

    plsc.subcore_barrier()
    moff = pl.multiple_of(s * ZROWS, 8)
    pltpu.sync_copy(acc.at[pl.ds(moff, ZROWS)],
                    msg_hbm.at[c, pl.ds(moff, ZROWS)])


UWH = UWP // 2


@functools.partial(
    pl.kernel,
    out_type=jax.ShapeDtypeStruct((E, UW), jnp.float32),
    mesh=_mesh,
    scratch_types=[
        pltpu.VMEM((2, 2, CHU), jnp.int32),
        pltpu.VMEM((2, CHU, UWP), jnp.bfloat16),
        pltpu.VMEM((2, CHU, UWP), jnp.bfloat16),
        pltpu.VMEM((2, CHU, UWP), jnp.bfloat16),
        pltpu.VMEM((2, CHU, UW), jnp.float32),
        pltpu.SemaphoreType.DMA,
        pltpu.SemaphoreType.DMA,
        pltpu.SemaphoreType.DMA,
        pltpu.SemaphoreType.DMA,
        pltpu.SemaphoreType.DMA,
    ],
    compiler_params=pltpu.CompilerParams(use_tc_tiling_on_sc=False,
                                         needs_layout_passes=False),
)
def _sc_edge_update(ei_hbm, p_hbm, q_hbm, r_hbm, out_hbm,
                    ei_v, p_v, q_v, r_v, o_v,
                    sem_i, sem_g0, sem_g1, sem_o0, sem_o1):
    c = lax.axis_index("c")
    s = lax.axis_index("s")
    base = (c * NS + s) * EPW

    def coff(j):
        jm = jnp.minimum(j, NCHU - 1)
        return pl.multiple_of(base + jm * CHU, 8)

    def issue_idx(j, b):
        pltpu.async_copy(ei_hbm.at[:, pl.ds(coff(j), CHU)], ei_v.at[b], sem_i)

    def wait_idx(b):
        pltpu.make_async_copy(ei_hbm.at[:, pl.ds(coff(0), CHU)],
                              ei_v.at[b], sem_i).wait()

    def issue_gathers(j, b, sem):
        pltpu.async_copy(p_hbm.at[ei_v.at[b, 0]], p_v.at[b], sem)
        pltpu.async_copy(q_hbm.at[ei_v.at[b, 1]], q_v.at[b], sem)
        pltpu.async_copy(r_hbm.at[pl.ds(coff(j), CHU)], r_v.at[b], sem)

    def wait_gathers(b, sem):
        for dst in (p_v, q_v, r_v):
            pltpu.make_async_copy(r_hbm.at[pl.ds(coff(0), CHU)],
                                  dst.at[b], sem).wait()

    def compute(b):
        lane = lax.iota(jnp.int32, 16)
        tail_mask = lane < 4

        def halves(v_ref, r, j):
            x = v_ref[b, r, pl.ds(j * 32, 32)]
            return plsc.unpack(x, format=plsc.PackFormat.INTERLEAVED)

        @plsc.parallel_loop(0, CHU, unroll=2)
        def _upd(r):
            for j in range(UWP // 32):
                plo, phi = halves(p_v, r, j)
                qlo, qhi = halves(q_v, r, j)
                rlo, rhi = halves(r_v, r, j)
                ylo = _sig(plo + qlo + rlo)
                yhi = _sig(phi + qhi + rhi)
                o_v[b, r, pl.ds(j * 32, 16)] = ylo
                if j < UWP // 32 - 1:
                    o_v[b, r, pl.ds(j * 32 + 16, 16)] = yhi
                else:
                    plsc.store_scatter(
                        o_v.at[b], [lane * 0 + r, j * 32 + 16 + lane],
                        yhi, mask=tail_mask)

    def issue_out(j, b, sem):
        pltpu.async_copy(o_v.at[b], out_hbm.at[pl.ds(coff(j), CHU)], sem)

    def wait_out(b, sem):
        pltpu.make_async_copy(o_v.at[b],
                              out_hbm.at[pl.ds(coff(0), CHU)], sem).wait()

    issue_idx(0, 0)
    wait_idx(0)
    issue_gathers(0, 0, sem_g0)
    issue_idx(1, 1)

    @pl.loop(0, NCHU2)
    def _pair(kk):
        j0 = kk * 2
        wait_idx(1)
        wait_gathers(0, sem_g0)
        issue_gathers(j0 + 1, 1, sem_g1)
        issue_idx(j0 + 2, 0)

        @pl.when(kk > 0)
        def _wo0():
            wait_out(0, sem_o0)

        compute(0)
        issue_out(j0, 0, sem_o0)

        wait_idx(0)
        wait_gathers(1, sem_g1)
        issue_gathers(j0 + 2, 0, sem_g0)
        issue_idx(j0 + 3, 1)

        @pl.when(kk > 0)
        def _wo1():
            wait_out(1, sem_o1)

        compute(1)
        issue_out(j0 + 1, 1, sem_o1)

    wait_idx(1)
    wait_gathers(0, sem_g0)
    wait_out(0, sem_o0)
    wait_out(1, sem_o1)


def kernel(node_features, edge_radial, edge_angular, edge_index,
           W1, b1, W2a, b2a, W2b, b2b, W2c, b2c, W2d, b2d,
           Ws, bs, Wn, bn, We, be):
    src = edge_index[0].astype(jnp.int32)
    dst = edge_index[1].astype(jnp.int32)

    zpad_s = jnp.zeros((SWP - SW,), jnp.float32)
    w1g = jnp.pad(W1[:ND], ((0, 0), (0, SWP - SW)))
    w1r = jnp.pad(W1[ND:ND + RAD], ((0, 0), (0, SWP - SW)))
    w1a = jnp.pad(W1[ND + RAD:], ((0, 0), (0, SWP - SW)))
    b1p = jnp.concatenate([b1, zpad_s])[None, :]
    wes = jnp.pad(We[:ND], ((0, 0), (0, UWP - UW)))[:, _PERM]
    wed = jnp.pad(We[ND:2 * ND], ((0, 0), (0, UWP - UW)))[:, _PERM]
    wer = jnp.pad(We[2 * ND:2 * ND + RAD], ((0, 0), (0, UWP - UW)))[:, _PERM]
    wea = jnp.pad(We[2 * ND + RAD:], ((0, 0), (0, UWP - UW)))[:, _PERM]
    bep = jnp.pad(be, (0, UWP - UW))[_PERM][None, :]

    g, p, q = _tables_call(node_features, w1g, wes, wed)
    s_t, r_t = _sr_call(edge_radial, edge_angular, w1r, w1a, b1p, wer, wea, bep)

    ei = edge_index.astype(jnp.int32)
    msg = _sc_messages(src, dst, g, s_t)
    edge_updated = _sc_edge_update(ei, p, q, r_t)

    node_updated = _node_call(
        msg, node_features, W2a, b2a[None, :], W2b, b2b[None, :],
        W2c, b2c[None, :], W2d, b2d[None, :], Ws, bs[None, :], Wn, bn[None, :])
    return (node_updated, edge_updated)


# speedup vs baseline: 1.6029x; 1.0400x over previous
import functools

import jax
import jax.numpy as jnp
import numpy as np
from jax import lax
from jax.experimental import pallas as pl
from jax.experimental.pallas import tpu as pltpu
from jax.experimental.pallas import tpu_sc as plsc

N = 10000
E = 320000
ND = 128
RAD = 16
ANG = 4
SW = 148
SWP = 160
UW = 276
UWP = 288
D1, D2, H = 90, 32, 64

NC, NS = 2, 16
NW = NC * NS
EPW = E // NW
CHM = 80
NCHM = EPW // CHM
CHU = 40
NCHU = EPW // CHU
NCHU2 = NCHU // 2
APAD = NW * 320
ZROWS = APAD // NS
RPT = N // NS


def _sig(x):
    return 1.0 / (1.0 + jnp.exp(-x))


_PERM = np.zeros(UWP, np.int32)
for _m in range(UWP // 32):
    for _k in range(16):
        _PERM[32 * _m + 2 * _k] = 32 * _m + _k
        _PERM[32 * _m + 2 * _k + 1] = 32 * _m + 16 + _k


def _tables_body(nf_ref, w1g_ref, wes_ref, wed_ref, g_ref, p_ref, q_ref):
    nf = nf_ref[...]
    g_ref[...] = jnp.dot(nf, w1g_ref[...], preferred_element_type=jnp.float32)
    p_ref[...] = jnp.dot(nf, wes_ref[...],
                         preferred_element_type=jnp.float32).astype(jnp.bfloat16)
    q_ref[...] = jnp.dot(nf, wed_ref[...],
                         preferred_element_type=jnp.float32).astype(jnp.bfloat16)


def _tables_call(nf, w1g, wes, wed):
    return pl.pallas_call(
        _tables_body,
        out_shape=(
            jax.ShapeDtypeStruct((N, SWP), jnp.float32),
            jax.ShapeDtypeStruct((N, UWP), jnp.bfloat16),
            jax.ShapeDtypeStruct((N, UWP), jnp.bfloat16),
        ),
    )(nf, w1g, wes, wed)


BE = 4000

def _sr_body(er_ref, ea_ref, w1r_ref, w1a_ref, b1_ref, wer_ref, wea_ref,
             be_ref, s_ref, r_ref):
    er = er_ref[...]
    ea = ea_ref[...]
    s_ref[...] = (jnp.dot(er, w1r_ref[...], preferred_element_type=jnp.float32)
                  + jnp.dot(ea, w1a_ref[...], preferred_element_type=jnp.float32)
                  + b1_ref[...])
    r_ref[...] = (jnp.dot(er, wer_ref[...], preferred_element_type=jnp.float32)
                  + jnp.dot(ea, wea_ref[...], preferred_element_type=jnp.float32)
                  + be_ref[...])


def _sr_call(er, ea, w1r, w1a, b1p, wer, wea, bep):
    nblk = E // BE
    return pl.pallas_call(
        _sr_body,
        grid=(nblk,),
        in_specs=[
            pl.BlockSpec((BE, RAD), lambda i: (i, 0)),
            pl.BlockSpec((BE, ANG), lambda i: (i, 0)),
            pl.BlockSpec((RAD, SWP), lambda i: (0, 0)),
            pl.BlockSpec((ANG, SWP), lambda i: (0, 0)),
            pl.BlockSpec((1, SWP), lambda i: (0, 0)),
            pl.BlockSpec((RAD, UWP), lambda i: (0, 0)),
            pl.BlockSpec((ANG, UWP), lambda i: (0, 0)),
            pl.BlockSpec((1, UWP), lambda i: (0, 0)),
        ],
        out_specs=(
            pl.BlockSpec((BE, SWP), lambda i: (i, 0)),
            pl.BlockSpec((BE, UWP), lambda i: (i, 0)),
        ),
        out_shape=(
            jax.ShapeDtypeStruct((E, SWP), jnp.float32),
            jax.ShapeDtypeStruct((E, UWP), jnp.float32),
        ),
    )(er, ea, w1r, w1a, b1p, wer, wea, bep)


def _node_body(msg_ref, nf_ref, w2a_ref, b2a_ref, w2b_ref, b2b_ref,
               w2c_ref, b2c_ref, w2d_ref, b2d_ref, ws_ref, bs_ref,
               wn_ref, bn_ref, out_ref):
    m = msg_ref[0][:N, :SW] + msg_ref[1][:N, :SW]
    h = jnp.maximum(jnp.dot(m, w2a_ref[...], preferred_element_type=jnp.float32)
                    + b2a_ref[...], 0.0)
    h = jnp.maximum(jnp.dot(h, w2b_ref[...], preferred_element_type=jnp.float32)
                    + b2b_ref[...], 0.0)
    h = jnp.maximum(jnp.dot(h, w2c_ref[...], preferred_element_type=jnp.float32)
                    + b2c_ref[...], 0.0)
    h = jnp.dot(h, w2d_ref[...], preferred_element_type=jnp.float32) + b2d_ref[...]
    nf = nf_ref[...]
    z = (jnp.dot(nf, ws_ref[...], preferred_element_type=jnp.float32) + bs_ref[...]
         + jnp.dot(h, wn_ref[...], preferred_element_type=jnp.float32) + bn_ref[...])
    out_ref[...] = _sig(z) + nf


def _node_call(msg, nf, w2a, b2a, w2b, b2b, w2c, b2c, w2d, b2d, ws, bs, wn, bn):
    return pl.pallas_call(
        _node_body,
        out_shape=jax.ShapeDtypeStruct((N, ND), jnp.float32),
    )(msg, nf, w2a, b2a, w2b, b2b, w2c, b2c, w2d, b2d, ws, bs, wn, bn)


_mesh = plsc.VectorSubcoreMesh(core_axis_name="c", subcore_axis_name="s")


@functools.partial(
    pl.kernel,
    out_type=jax.ShapeDtypeStruct((NC, APAD, SWP), jnp.float32),
    mesh=_mesh,
    scratch_types=[
        pltpu.VMEM((CHM,), jnp.int32),
        pltpu.VMEM((CHM,), jnp.int32),
        pltpu.VMEM((CHM,), jnp.int32),
        pltpu.VMEM((CHM, SWP), jnp.float32),
        pltpu.VMEM((CHM, SWP), jnp.float32),
        pltpu.VMEM_SHARED((APAD, SWP), jnp.float32),
        pltpu.SemaphoreType.DMA,
    ],
    compiler_params=pltpu.CompilerParams(use_tc_tiling_on_sc=False,
                                         needs_layout_passes=False),
)
def _sc_messages(src_hbm, dst_hbm, g_hbm, s_hbm, msg_hbm,
                 src_idx, dst_idx, src2_idx,
                 g_v, s_v, acc, sem_a):
    c = lax.axis_index("c")
    s = lax.axis_index("s")
    wid = c * NS + s
    base = wid * EPW

    @pl.loop(0, CHM)
    def _zrow(r):
        for j in range(SWP // 16):
            s_v[r, pl.ds(j * 16, 16)] = jnp.zeros((16,), jnp.float32)

    for z in range(ZROWS // CHM):
        zoff = pl.multiple_of(s * ZROWS + z * CHM, 8)
        pltpu.sync_copy(s_v, acc.at[pl.ds(zoff, CHM)])

    plsc.subcore_barrier()

    @pl.loop(0, NCHM)
    def _chunk(k):
        off = pl.multiple_of(base + k * CHM, 8)
        pltpu.sync_copy(src_hbm.at[pl.ds(off, CHM)], src_idx)
        pltpu.sync_copy(dst_hbm.at[pl.ds(off, CHM)], dst_idx)
        pltpu.async_copy(src_hbm.at[src_idx], src2_idx, sem_a).wait()
        cp_g = pltpu.async_copy(g_hbm.at[src2_idx], g_v, sem_a)
        cp_s = pltpu.async_copy(s_hbm.at[pl.ds(off, CHM)], s_v, sem_a)
        cp_g.wait()
        cp_s.wait()

        @plsc.parallel_loop(0, CHM, unroll=2)
        def _sub(r):
            for j in range(SWP // 16):
                x = g_v[r, pl.ds(j * 16, 16)] + s_v[r, pl.ds(j * 16, 16)]
                s_v[r, pl.ds(j * 16, 16)] = _sig(x)

        pltpu.sync_copy(s_v, acc.at[dst_idx], add=True)

    plsc.subcore_barrier()
    moff = pl.multiple_of(s * ZROWS, 8)
    pltpu.sync_copy(acc.at[pl.ds(moff, ZROWS)],
                    msg_hbm.at[c, pl.ds(moff, ZROWS)])


UWH = UWP // 2


@functools.partial(
    pl.kernel,
    out_type=jax.ShapeDtypeStruct((E, UW), jnp.float32),
    mesh=_mesh,
    scratch_types=[
        pltpu.VMEM((2, 2, CHU), jnp.int32),
        pltpu.VMEM((2, CHU, UWP), jnp.bfloat16),
        pltpu.VMEM((2, CHU, UWP), jnp.bfloat16),
        pltpu.VMEM((2, CHU, UWP), jnp.float32),
        pltpu.VMEM((2, CHU, UW), jnp.float32),
        pltpu.SemaphoreType.DMA,
        pltpu.SemaphoreType.DMA,
        pltpu.SemaphoreType.DMA,
        pltpu.SemaphoreType.DMA,
        pltpu.SemaphoreType.DMA,
    ],
    compiler_params=pltpu.CompilerParams(use_tc_tiling_on_sc=False,
                                         needs_layout_passes=False),
)
def _sc_edge_update(ei_hbm, p_hbm, q_hbm, r_hbm, out_hbm,
                    ei_v, p_v, q_v, r_v, o_v,
                    sem_i, sem_g0, sem_g1, sem_o0, sem_o1):
    c = lax.axis_index("c")
    s = lax.axis_index("s")
    base = (c * NS + s) * EPW

    def coff(j):
        jm = jnp.minimum(j, NCHU - 1)
        return pl.multiple_of(base + jm * CHU, 8)

    def issue_idx(j, b):
        pltpu.async_copy(ei_hbm.at[:, pl.ds(coff(j), CHU)], ei_v.at[b], sem_i)

    def wait_idx(b):
        pltpu.make_async_copy(ei_hbm.at[:, pl.ds(coff(0), CHU)],
                              ei_v.at[b], sem_i).wait()

    def issue_gathers(j, b, sem):
        pltpu.async_copy(p_hbm.at[ei_v.at[b, 0]], p_v.at[b], sem)
        pltpu.async_copy(q_hbm.at[ei_v.at[b, 1]], q_v.at[b], sem)
        pltpu.async_copy(r_hbm.at[pl.ds(coff(j), CHU)], r_v.at[b], sem)

    def wait_gathers(b, sem):
        for dst in (p_v, q_v, r_v):
            pltpu.make_async_copy(r_hbm.at[pl.ds(coff(0), CHU)],
                                  dst.at[b], sem).wait()

    def compute(b):
        lane = lax.iota(jnp.int32, 16)
        tail_mask = lane < 4

        def halves(v_ref, r, j):
            x = v_ref[b, r, pl.ds(j * 32, 32)]
            return plsc.unpack(x, format=plsc.PackFormat.INTERLEAVED)

        @plsc.parallel_loop(0, CHU, unroll=2)
        def _upd(r):
            for j in range(UWP // 32):
                plo, phi = halves(p_v, r, j)
                qlo, qhi = halves(q_v, r, j)
                rlo = r_v[b, r, pl.ds(j * 32, 16)]
                rhi = r_v[b, r, pl.ds(j * 32 + 16, 16)]
                ylo = _sig(plo + qlo + rlo)
                yhi = _sig(phi + qhi + rhi)
                o_v[b, r, pl.ds(j * 32, 16)] = ylo
                if j < UWP // 32 - 1:
                    o_v[b, r, pl.ds(j * 32 + 16, 16)] = yhi
                else:
                    plsc.store_scatter(
                        o_v.at[b], [lane * 0 + r, j * 32 + 16 + lane],
                        yhi, mask=tail_mask)

    def issue_out(j, b, sem):
        pltpu.async_copy(o_v.at[b], out_hbm.at[pl.ds(coff(j), CHU)], sem)

    def wait_out(b, sem):
        pltpu.make_async_copy(o_v.at[b],
                              out_hbm.at[pl.ds(coff(0), CHU)], sem).wait()

    issue_idx(0, 0)
    wait_idx(0)
    issue_gathers(0, 0, sem_g0)
    issue_idx(1, 1)

    @pl.loop(0, NCHU2)
    def _pair(kk):
        j0 = kk * 2
        wait_idx(1)
        wait_gathers(0, sem_g0)
        issue_gathers(j0 + 1, 1, sem_g1)
        issue_idx(j0 + 2, 0)

        @pl.when(kk > 0)
        def _wo0():
            wait_out(0, sem_o0)

        compute(0)
        issue_out(j0, 0, sem_o0)

        wait_idx(0)
        wait_gathers(1, sem_g1)
        issue_gathers(j0 + 2, 0, sem_g0)
        issue_idx(j0 + 3, 1)

        @pl.when(kk > 0)
        def _wo1():
            wait_out(1, sem_o1)

        compute(1)
        issue_out(j0 + 1, 1, sem_o1)

    wait_idx(1)
    wait_gathers(0, sem_g0)
    wait_out(0, sem_o0)
    wait_out(1, sem_o1)


def kernel(node_features, edge_radial, edge_angular, edge_index,
           W1, b1, W2a, b2a, W2b, b2b, W2c, b2c, W2d, b2d,
           Ws, bs, Wn, bn, We, be):
    src = edge_index[0].astype(jnp.int32)
    dst = edge_index[1].astype(jnp.int32)

    zpad_s = jnp.zeros((SWP - SW,), jnp.float32)
    w1g = jnp.pad(W1[:ND], ((0, 0), (0, SWP - SW)))
    w1r = jnp.pad(W1[ND:ND + RAD], ((0, 0), (0, SWP - SW)))
    w1a = jnp.pad(W1[ND + RAD:], ((0, 0), (0, SWP - SW)))
    b1p = jnp.concatenate([b1, zpad_s])[None, :]
    wes = jnp.pad(We[:ND], ((0, 0), (0, UWP - UW)))[:, _PERM]
    wed = jnp.pad(We[ND:2 * ND], ((0, 0), (0, UWP - UW)))[:, _PERM]
    wer = jnp.pad(We[2 * ND:2 * ND + RAD], ((0, 0), (0, UWP - UW)))
    wea = jnp.pad(We[2 * ND + RAD:], ((0, 0), (0, UWP - UW)))
    bep = jnp.pad(be, (0, UWP - UW))[None, :]

    g, p, q = _tables_call(node_features, w1g, wes, wed)
    s_t, r_t = _sr_call(edge_radial, edge_angular, w1r, w1a, b1p, wer, wea, bep)

    ei = edge_index.astype(jnp.int32)
    msg = _sc_messages(src, dst, g, s_t)
    edge_updated = _sc_edge_update(ei, p, q, r_t)

    node_updated = _node_call(
        msg, node_features, W2a, b2a[None, :], W2b, b2b[None, :],
        W2c, b2c[None, :], W2d, b2d[None, :], Ws, bs[None, :], Wn, bn[None, :])
    return (node_updated, edge_updated)
